# Initial kernel scaffold; baseline (speedup 1.0000x reference)
import functools

import jax
import jax.numpy as jnp
from jax import lax
from jax.experimental import pallas as pl
from jax.experimental.pallas import tpu as pltpu
from jax.experimental.pallas import tpu_sc as plsc

_NC = 2
_NS = 16
_NW = _NC * _NS
_CHUNK = 128
_HID = 32
_DEGW = 16


def _deg_body(k_chunks, n_pad, dstw, ones_hbm, zeros_hbm, out_hbm,
              idxd, ones_v, acc):
    c = lax.axis_index("c")
    s = lax.axis_index("s")
    w = c * _NS + s
    rpt = n_pad // _NS
    pltpu.sync_copy(zeros_hbm.at[pl.ds(s * rpt, rpt)],
                    acc.at[pl.ds(s * rpt, rpt)])
    pltpu.sync_copy(dstw.at[w], idxd)
    pltpu.sync_copy(ones_hbm, ones_v)
    plsc.subcore_barrier()

    def body(j, carry):
        pltpu.sync_copy(ones_v, acc.at[idxd.at[j]], add=True)
        return carry

    lax.fori_loop(0, k_chunks, body, 0)
    plsc.subcore_barrier()
    pltpu.sync_copy(acc.at[pl.ds(s * rpt, rpt)],
                    out_hbm.at[c, pl.ds(s * rpt, rpt)])


def _edge_body(k_chunks, n_pad, g_hbm, srcw, dstw, zeros_hbm, out_hbm,
               idxs, idxd, rows, acc, sem):
    c = lax.axis_index("c")
    s = lax.axis_index("s")
    w = c * _NS + s
    rpt = n_pad // _NS
    pltpu.sync_copy(zeros_hbm.at[pl.ds(s * rpt, rpt)],
                    acc.at[pl.ds(s * rpt, rpt)])
    pltpu.sync_copy(srcw.at[w], idxs)
    pltpu.sync_copy(dstw.at[w], idxd)
    plsc.subcore_barrier()

    def body(j, carry):
        pltpu.async_copy(g_hbm.at[idxs.at[j]], rows, sem).wait()
        pltpu.sync_copy(rows, acc.at[idxd.at[j]], add=True)
        return carry

    lax.fori_loop(0, k_chunks, body, 0)
    plsc.subcore_barrier()
    pltpu.sync_copy(acc.at[pl.ds(s * rpt, rpt)],
                    out_hbm.at[c, pl.ds(s * rpt, rpt)])


def _dense0_body(degp_ref, x_ref, w0_ref, dinv_ref, g_ref):
    n = x_ref.shape[0]
    deg = degp_ref[0, 0:n, 0:1] + degp_ref[1, 0:n, 0:1] + 1.0
    dinv = lax.rsqrt(jnp.maximum(deg, 1.0))
    dinv_b = jnp.broadcast_to(dinv, (n, _HID))
    dinv_ref[...] = dinv_b
    g_ref[...] = jnp.dot(x_ref[...], w0_ref[...],
                         preferred_element_type=jnp.float32) * dinv_b


def _mid_body(p_ref, gprev_ref, dinv_ref, w_ref, b_ref, g_ref):
    n = gprev_ref.shape[0]
    dinv = dinv_ref[...]
    agg = p_ref[0, 0:n, :] + p_ref[1, 0:n, :] + gprev_ref[...]
    h = jnp.maximum(dinv * agg + b_ref[...], 0.0)
    g_ref[...] = jnp.dot(h, w_ref[...],
                         preferred_element_type=jnp.float32) * dinv


def _final_body(p_ref, gprev_ref, dinv_ref, b_ref, batch_ref, linw_ref,
                linb_ref, out_ref):
    n = gprev_ref.shape[0]
    ng = out_ref.shape[0]
    agg = p_ref[0, 0:n, :] + p_ref[1, 0:n, :] + gprev_ref[...]
    h = jnp.maximum(dinv_ref[...] * agg + b_ref[...], 0.0)
    gid = lax.broadcasted_iota(jnp.int32, (ng, n), 0)
    onehot = (batch_ref[...] == gid).astype(jnp.float32)
    summed = jnp.dot(onehot, h, preferred_element_type=jnp.float32)
    counts = jnp.sum(onehot, axis=1, keepdims=True)
    pooled = summed / jnp.maximum(counts, 1.0)
    out_ref[...] = (jnp.dot(pooled, linw_ref[...],
                            preferred_element_type=jnp.float32)
                    + linb_ref[...])


def kernel(x, edge_index, batch, W0, b0, W1, b1, W2, b2, W3, b3,
           lin_W, lin_b):
    n = x.shape[0]
    e = edge_index.shape[1]
    ng = 64
    n_pad = (n // _NS + 1) * _NS
    k_chunks = -(-e // (_NW * _CHUNK))
    e_pad = k_chunks * _NW * _CHUNK

    src = edge_index[0].astype(jnp.int32)
    dst = edge_index[1].astype(jnp.int32)
    if e_pad > e:
        src = jnp.concatenate([src, jnp.zeros((e_pad - e,), jnp.int32)])
        dst = jnp.concatenate([dst, jnp.full((e_pad - e,), n, jnp.int32)])
    srcw = src.reshape(_NW, k_chunks, _CHUNK)
    dstw = dst.reshape(_NW, k_chunks, _CHUNK)
    zeros_h = jnp.zeros((n_pad, _HID), jnp.float32)
    zeros_d = jnp.zeros((n_pad, _DEGW), jnp.float32)
    ones_d = jnp.ones((_CHUNK, _DEGW), jnp.float32)
    batch2 = batch.astype(jnp.int32).reshape(1, n)

    mesh = plsc.VectorSubcoreMesh(core_axis_name="c", subcore_axis_name="s")
    deg_call = pl.kernel(
        functools.partial(_deg_body, k_chunks, n_pad),
        out_type=jax.ShapeDtypeStruct((_NC, n_pad, _DEGW), jnp.float32),
        mesh=mesh,
        scratch_types=[
            pltpu.VMEM((k_chunks, _CHUNK), jnp.int32),
            pltpu.VMEM((_CHUNK, _DEGW), jnp.float32),
            pltpu.VMEM_SHARED((n_pad, _DEGW), jnp.float32),
        ],
    )
    edge_call = pl.kernel(
        functools.partial(_edge_body, k_chunks, n_pad),
        out_type=jax.ShapeDtypeStruct((_NC, n_pad, _HID), jnp.float32),
        mesh=mesh,
        scratch_types=[
            pltpu.VMEM((k_chunks, _CHUNK), jnp.int32),
            pltpu.VMEM((k_chunks, _CHUNK), jnp.int32),
            pltpu.VMEM((_CHUNK, _HID), jnp.float32),
            pltpu.VMEM_SHARED((n_pad, _HID), jnp.float32),
            pltpu.SemaphoreType.DMA,
        ],
    )

    degp = deg_call(dstw, ones_d, zeros_d)

    dinv, g = pl.pallas_call(
        _dense0_body,
        out_shape=(jax.ShapeDtypeStruct((n, _HID), jnp.float32),
                   jax.ShapeDtypeStruct((n, _HID), jnp.float32)),
    )(degp, x, W0)

    for (b_prev, w_next) in ((b0, W1), (b1, W2), (b2, W3)):
        p = edge_call(g, srcw, dstw, zeros_h)
        g = pl.pallas_call(
            _mid_body,
            out_shape=jax.ShapeDtypeStruct((n, _HID), jnp.float32),
        )(p, g, dinv, w_next, b_prev.reshape(1, _HID))

    p = edge_call(g, srcw, dstw, zeros_h)
    out = pl.pallas_call(
        _final_body,
        out_shape=jax.ShapeDtypeStruct((ng, 1), jnp.float32),
    )(p, g, dinv, b3.reshape(1, _HID), batch2, lin_W, lin_b.reshape(1, 1))
    return jnp.squeeze(out, -1)


# speedup vs baseline: 20.3944x; 20.3944x over previous
import functools

import jax
import jax.numpy as jnp
from jax import lax
from jax.experimental import pallas as pl
from jax.experimental.pallas import tpu as pltpu
from jax.experimental.pallas import tpu_sc as plsc

_NC = 2
_NS = 16
_NW = _NC * _NS
_CHUNK = 128
_HID = 32
_DEGW = 16


def _deg_body(k_chunks, n_pad, dstw, ones_hbm, zeros_hbm, out_hbm,
              idxd, ones_v, acc):
    c = lax.axis_index("c")
    s = lax.axis_index("s")
    w = c * _NS + s
    rpt = n_pad // _NS
    pltpu.sync_copy(zeros_hbm.at[pl.ds(s * rpt, rpt)],
                    acc.at[pl.ds(s * rpt, rpt)])
    pltpu.sync_copy(dstw.at[w], idxd)
    pltpu.sync_copy(ones_hbm, ones_v)
    plsc.subcore_barrier()

    def body(j, carry):
        pltpu.sync_copy(ones_v, acc.at[idxd.at[j]], add=True)
        return carry

    lax.fori_loop(0, k_chunks, body, 0)
    plsc.subcore_barrier()
    pltpu.sync_copy(acc.at[pl.ds(s * rpt, rpt)],
                    out_hbm.at[c, pl.ds(s * rpt, rpt)])


def _edge_body(k_chunks, n_pad, g_hbm, srcw, dstw, zeros_hbm, out_hbm,
               idxs, idxd, rows, acc, sem):
    c = lax.axis_index("c")
    s = lax.axis_index("s")
    w = c * _NS + s
    rpt = n_pad // _NS
    pltpu.sync_copy(zeros_hbm.at[pl.ds(s * rpt, rpt)],
                    acc.at[pl.ds(s * rpt, rpt)])
    pltpu.sync_copy(srcw.at[w], idxs)
    pltpu.sync_copy(dstw.at[w], idxd)
    plsc.subcore_barrier()

    def body(j, carry):
        pltpu.async_copy(g_hbm.at[idxs.at[j]], rows, sem).wait()
        pltpu.sync_copy(rows, acc.at[idxd.at[j]], add=True)
        return carry

    lax.fori_loop(0, k_chunks, body, 0)
    plsc.subcore_barrier()
    pltpu.sync_copy(acc.at[pl.ds(s * rpt, rpt)],
                    out_hbm.at[c, pl.ds(s * rpt, rpt)])


def _dense0_body(degp_ref, x_ref, w0_ref, dinv_ref, g_ref):
    n = x_ref.shape[0]
    deg = degp_ref[0, 0:n, 0:1] + degp_ref[1, 0:n, 0:1] + 1.0
    dinv = lax.rsqrt(jnp.maximum(deg, 1.0))
    dinv_b = jnp.broadcast_to(dinv, (n, _HID))
    dinv_ref[...] = dinv_b
    g_ref[...] = jnp.dot(x_ref[...], w0_ref[...],
                         preferred_element_type=jnp.float32) * dinv_b


def _mid_body(p_ref, gprev_ref, dinv_ref, w_ref, b_ref, g_ref):
    n = gprev_ref.shape[0]
    dinv = dinv_ref[...]
    agg = p_ref[0, 0:n, :] + p_ref[1, 0:n, :] + gprev_ref[...]
    h = jnp.maximum(dinv * agg + b_ref[...], 0.0)
    g_ref[...] = jnp.dot(h, w_ref[...],
                         preferred_element_type=jnp.float32) * dinv


def _final_body(p_ref, gprev_ref, dinv_ref, b_ref, batch_ref, linw_ref,
                linb_ref, out_ref):
    n = gprev_ref.shape[0]
    ng = out_ref.shape[0]
    agg = p_ref[0, 0:n, :] + p_ref[1, 0:n, :] + gprev_ref[...]
    h = jnp.maximum(dinv_ref[...] * agg + b_ref[...], 0.0)
    gid = lax.broadcasted_iota(jnp.int32, (ng, n), 0)
    onehot = (batch_ref[...] == gid).astype(jnp.float32)
    summed = jnp.dot(onehot, h, preferred_element_type=jnp.float32)
    counts = jnp.sum(onehot, axis=1, keepdims=True)
    pooled = summed / jnp.maximum(counts, 1.0)
    out_ref[...] = (jnp.dot(pooled, linw_ref[...],
                            preferred_element_type=jnp.float32)
                    + linb_ref[...])


def kernel(x, edge_index, batch, W0, b0, W1, b1, W2, b2, W3, b3,
           lin_W, lin_b):
    n = x.shape[0]
    e = edge_index.shape[1]
    ng = 64
    n_pad = ((n + 1 + 127) // 128) * 128
    k_chunks = -(-e // (_NW * _CHUNK))
    e_pad = k_chunks * _NW * _CHUNK

    src = edge_index[0].astype(jnp.int32)
    dst = edge_index[1].astype(jnp.int32)
    if e_pad > e:
        src = jnp.concatenate([src, jnp.zeros((e_pad - e,), jnp.int32)])
        dst = jnp.concatenate([dst, jnp.full((e_pad - e,), n, jnp.int32)])
    srcw = src.reshape(_NW, k_chunks, _CHUNK)
    dstw = dst.reshape(_NW, k_chunks, _CHUNK)
    zeros_h = jnp.zeros((n_pad, _HID), jnp.float32)
    zeros_d = jnp.zeros((n_pad, _DEGW), jnp.float32)
    ones_d = jnp.ones((_CHUNK, _DEGW), jnp.float32)
    batch2 = batch.astype(jnp.int32).reshape(1, n)

    mesh = plsc.VectorSubcoreMesh(core_axis_name="c", subcore_axis_name="s")
    sc_params = pltpu.CompilerParams(use_tc_tiling_on_sc=False)
    deg_call = pl.kernel(
        functools.partial(_deg_body, k_chunks, n_pad),
        out_type=jax.ShapeDtypeStruct((_NC, n_pad, _DEGW), jnp.float32),
        mesh=mesh,
        scratch_types=[
            pltpu.VMEM((k_chunks, _CHUNK), jnp.int32),
            pltpu.VMEM((_CHUNK, _DEGW), jnp.float32),
            pltpu.VMEM_SHARED((n_pad, _DEGW), jnp.float32),
        ],
        compiler_params=sc_params,
    )
    edge_call = pl.kernel(
        functools.partial(_edge_body, k_chunks, n_pad),
        out_type=jax.ShapeDtypeStruct((_NC, n_pad, _HID), jnp.float32),
        mesh=mesh,
        scratch_types=[
            pltpu.VMEM((k_chunks, _CHUNK), jnp.int32),
            pltpu.VMEM((k_chunks, _CHUNK), jnp.int32),
            pltpu.VMEM((_CHUNK, _HID), jnp.float32),
            pltpu.VMEM_SHARED((n_pad, _HID), jnp.float32),
            pltpu.SemaphoreType.DMA,
        ],
        compiler_params=sc_params,
    )

    degp = deg_call(dstw, ones_d, zeros_d)

    dinv, g = pl.pallas_call(
        _dense0_body,
        out_shape=(jax.ShapeDtypeStruct((n, _HID), jnp.float32),
                   jax.ShapeDtypeStruct((n, _HID), jnp.float32)),
    )(degp, x, W0)

    for (b_prev, w_next) in ((b0, W1), (b1, W2), (b2, W3)):
        p = edge_call(g, srcw, dstw, zeros_h)
        g = pl.pallas_call(
            _mid_body,
            out_shape=jax.ShapeDtypeStruct((n, _HID), jnp.float32),
        )(p, g, dinv, w_next, b_prev.reshape(1, _HID))

    p = edge_call(g, srcw, dstw, zeros_h)
    out = pl.pallas_call(
        _final_body,
        out_shape=jax.ShapeDtypeStruct((ng, 1), jnp.float32),
    )(p, g, dinv, b3.reshape(1, _HID), batch2, lin_W, lin_b.reshape(1, 1))
    return jnp.squeeze(out, -1)


# speedup vs baseline: 22.3927x; 1.0980x over previous
import functools

import jax
import jax.numpy as jnp
from jax import lax
from jax.experimental import pallas as pl
from jax.experimental.pallas import tpu as pltpu
from jax.experimental.pallas import tpu_sc as plsc

_NC = 2
_NS = 16
_NW = _NC * _NS
_CHUNK = 128
_HID = 32
_DEGW = 16
_NBUF = 8
_PREF = 4


def _deg_body(k_chunks, n_pad, dstw, ones_hbm, zeros_hbm, out_hbm,
              idxd, ones_v, acc):
    c = lax.axis_index("c")
    s = lax.axis_index("s")
    w = c * _NS + s
    rpt = n_pad // _NS
    pltpu.sync_copy(zeros_hbm.at[pl.ds(s * rpt, rpt)],
                    acc.at[pl.ds(s * rpt, rpt)])
    pltpu.sync_copy(dstw.at[w], idxd)
    pltpu.sync_copy(ones_hbm, ones_v)
    plsc.subcore_barrier()

    def body(j, carry):
        pltpu.sync_copy(ones_v, acc.at[idxd.at[j]], add=True)
        return carry

    lax.fori_loop(0, k_chunks, body, 0)
    plsc.subcore_barrier()
    pltpu.sync_copy(acc.at[pl.ds(s * rpt, rpt)],
                    out_hbm.at[c, pl.ds(s * rpt, rpt)])


def _edge_body(k_groups, n_pad, g_hbm, srcw, dstw, zeros_hbm, out_hbm,
               idxs, idxd, rows, acc, gsem, ssem):
    c = lax.axis_index("c")
    s = lax.axis_index("s")
    w = c * _NS + s
    rpt = n_pad // _NS
    pltpu.sync_copy(zeros_hbm.at[pl.ds(s * rpt, rpt)],
                    acc.at[pl.ds(s * rpt, rpt)])
    pltpu.sync_copy(srcw.at[w], idxs)
    pltpu.sync_copy(dstw.at[w], idxd)
    plsc.subcore_barrier()

    def start_g(j, b):
        pltpu.async_copy(g_hbm.at[idxs.at[j]], rows.at[b], gsem.at[b])

    def wait_g(b):
        pltpu.make_async_copy(g_hbm.at[idxs.at[0]], rows.at[b],
                              gsem.at[b]).wait()

    def start_s(j, b):
        pltpu.async_copy(rows.at[b], acc.at[idxd.at[j]], ssem.at[b],
                         add=True)

    def wait_s(b):
        pltpu.make_async_copy(rows.at[b], acc.at[idxd.at[0]],
                              ssem.at[b]).wait()

    for b in range(_PREF):
        start_g(b, b)

    def body(j0, carry):
        for b in range(_NBUF):
            j = j0 * _NBUF + b
            wait_g(b)
            start_s(j, b)
            jn = j + _PREF
            bn = (b + _PREF) % _NBUF
            if b < _PREF:
                @pl.when(j0 > 0)
                def _():
                    wait_s(bn)
                start_g(jn, bn)
            else:
                @pl.when(j0 + 1 < k_groups)
                def _():
                    wait_s(bn)
                    start_g(jn, bn)
        return carry

    lax.fori_loop(0, k_groups, body, 0)
    for b in range(_NBUF):
        wait_s(b)
    plsc.subcore_barrier()
    pltpu.sync_copy(acc.at[pl.ds(s * rpt, rpt)],
                    out_hbm.at[c, pl.ds(s * rpt, rpt)])


def _dense0_body(degp_ref, x_ref, w0_ref, dinv_ref, g_ref):
    n = x_ref.shape[0]
    deg = degp_ref[0, 0:n, 0:1] + degp_ref[1, 0:n, 0:1] + 1.0
    dinv = lax.rsqrt(jnp.maximum(deg, 1.0))
    dinv_b = jnp.broadcast_to(dinv, (n, _HID))
    dinv_ref[...] = dinv_b
    g_ref[...] = jnp.dot(x_ref[...], w0_ref[...],
                         preferred_element_type=jnp.float32) * dinv_b


def _mid_body(p_ref, gprev_ref, dinv_ref, w_ref, b_ref, g_ref):
    n = gprev_ref.shape[0]
    dinv = dinv_ref[...]
    agg = p_ref[0, 0:n, :] + p_ref[1, 0:n, :] + gprev_ref[...]
    h = jnp.maximum(dinv * agg + b_ref[...], 0.0)
    g_ref[...] = jnp.dot(h, w_ref[...],
                         preferred_element_type=jnp.float32) * dinv


def _final_body(p_ref, gprev_ref, dinv_ref, b_ref, batch_ref, linw_ref,
                linb_ref, out_ref):
    n = gprev_ref.shape[0]
    ng = out_ref.shape[0]
    agg = p_ref[0, 0:n, :] + p_ref[1, 0:n, :] + gprev_ref[...]
    h = jnp.maximum(dinv_ref[...] * agg + b_ref[...], 0.0)
    gid = lax.broadcasted_iota(jnp.int32, (ng, n), 0)
    onehot = (batch_ref[...] == gid).astype(jnp.float32)
    summed = jnp.dot(onehot, h, preferred_element_type=jnp.float32)
    counts = jnp.sum(onehot, axis=1, keepdims=True)
    pooled = summed / jnp.maximum(counts, 1.0)
    out_ref[...] = (jnp.dot(pooled, linw_ref[...],
                            preferred_element_type=jnp.float32)
                    + linb_ref[...])


def kernel(x, edge_index, batch, W0, b0, W1, b1, W2, b2, W3, b3,
           lin_W, lin_b):
    n = x.shape[0]
    e = edge_index.shape[1]
    ng = 64
    n_pad = ((n + 1 + 127) // 128) * 128
    k_chunks = -(-e // (_NW * _CHUNK))
    k_groups = -(-k_chunks // _NBUF)
    k_chunks = k_groups * _NBUF
    e_pad = k_chunks * _NW * _CHUNK

    src = edge_index[0].astype(jnp.int32)
    dst = edge_index[1].astype(jnp.int32)
    if e_pad > e:
        src = jnp.concatenate([src, jnp.zeros((e_pad - e,), jnp.int32)])
        dst = jnp.concatenate([dst, jnp.full((e_pad - e,), n, jnp.int32)])
    srcw = src.reshape(_NW, k_chunks, _CHUNK)
    dstw = dst.reshape(_NW, k_chunks, _CHUNK)
    zeros_h = jnp.zeros((n_pad, _HID), jnp.float32)
    zeros_d = jnp.zeros((n_pad, _DEGW), jnp.float32)
    ones_d = jnp.ones((_CHUNK, _DEGW), jnp.float32)
    batch2 = batch.astype(jnp.int32).reshape(1, n)

    mesh = plsc.VectorSubcoreMesh(core_axis_name="c", subcore_axis_name="s")
    sc_params = pltpu.CompilerParams(use_tc_tiling_on_sc=False)
    deg_call = pl.kernel(
        functools.partial(_deg_body, k_chunks, n_pad),
        out_type=jax.ShapeDtypeStruct((_NC, n_pad, _DEGW), jnp.float32),
        mesh=mesh,
        scratch_types=[
            pltpu.VMEM((k_chunks, _CHUNK), jnp.int32),
            pltpu.VMEM((_CHUNK, _DEGW), jnp.float32),
            pltpu.VMEM_SHARED((n_pad, _DEGW), jnp.float32),
        ],
        compiler_params=sc_params,
    )
    edge_call = pl.kernel(
        functools.partial(_edge_body, k_groups, n_pad),
        out_type=jax.ShapeDtypeStruct((_NC, n_pad, _HID), jnp.float32),
        mesh=mesh,
        scratch_types=[
            pltpu.VMEM((k_chunks, _CHUNK), jnp.int32),
            pltpu.VMEM((k_chunks, _CHUNK), jnp.int32),
            pltpu.VMEM((_NBUF, _CHUNK, _HID), jnp.float32),
            pltpu.VMEM_SHARED((n_pad, _HID), jnp.float32),
            pltpu.SemaphoreType.DMA((_NBUF,)),
            pltpu.SemaphoreType.DMA((_NBUF,)),
        ],
        compiler_params=sc_params,
    )

    degp = deg_call(dstw, ones_d, zeros_d)

    dinv, g = pl.pallas_call(
        _dense0_body,
        out_shape=(jax.ShapeDtypeStruct((n, _HID), jnp.float32),
                   jax.ShapeDtypeStruct((n, _HID), jnp.float32)),
    )(degp, x, W0)

    for (b_prev, w_next) in ((b0, W1), (b1, W2), (b2, W3)):
        p = edge_call(g, srcw, dstw, zeros_h)
        g = pl.pallas_call(
            _mid_body,
            out_shape=jax.ShapeDtypeStruct((n, _HID), jnp.float32),
        )(p, g, dinv, w_next, b_prev.reshape(1, _HID))

    p = edge_call(g, srcw, dstw, zeros_h)
    out = pl.pallas_call(
        _final_body,
        out_shape=jax.ShapeDtypeStruct((ng, 1), jnp.float32),
    )(p, g, dinv, b3.reshape(1, _HID), batch2, lin_W, lin_b.reshape(1, 1))
    return jnp.squeeze(out, -1)


# speedup vs baseline: 42.5051x; 1.8982x over previous
import functools

import jax
import jax.numpy as jnp
from jax import lax
from jax.experimental import pallas as pl
from jax.experimental.pallas import tpu as pltpu
from jax.experimental.pallas import tpu_sc as plsc

_NC = 2
_NS = 16
_NW = _NC * _NS
_CHUNK = 128
_HID = 32
_DEGW = 16
_NBUF = 8
_PREF = 4


def _deg_body(k_chunks, n_pad, dstw, ones_hbm, zeros_hbm, out_hbm,
              idxd, ones_v, acc):
    c = lax.axis_index("c")
    s = lax.axis_index("s")
    w = c * _NS + s
    rpt = n_pad // _NS
    pltpu.sync_copy(zeros_hbm.at[pl.ds(s * rpt, rpt)],
                    acc.at[pl.ds(s * rpt, rpt)])
    pltpu.sync_copy(dstw.at[w], idxd)
    pltpu.sync_copy(ones_hbm, ones_v)
    plsc.subcore_barrier()

    def body(j, carry):
        pltpu.sync_copy(ones_v, acc.at[idxd.at[j]], add=True)
        return carry

    lax.fori_loop(0, k_chunks, body, 0)
    plsc.subcore_barrier()
    pltpu.sync_copy(acc.at[pl.ds(s * rpt, rpt)],
                    out_hbm.at[c, pl.ds(s * rpt, rpt)])


def _edge_body(k_groups, n_pad, g_hbm, srcw, dstw, zeros_hbm, out_hbm,
               idxs, idxd, rows, gsrc, acc, gsem, ssem):
    c = lax.axis_index("c")
    s = lax.axis_index("s")
    w = c * _NS + s
    rpt = n_pad // _NS
    pltpu.sync_copy(zeros_hbm.at[pl.ds(s * rpt, rpt)],
                    acc.at[pl.ds(s * rpt, rpt)])
    pltpu.sync_copy(g_hbm.at[pl.ds(s * rpt, rpt)],
                    gsrc.at[pl.ds(s * rpt, rpt)])
    pltpu.sync_copy(srcw.at[w], idxs)
    pltpu.sync_copy(dstw.at[w], idxd)
    plsc.subcore_barrier()

    def start_g(j, b):
        pltpu.async_copy(gsrc.at[idxs.at[j]], rows.at[b], gsem.at[b])

    def wait_g(b):
        pltpu.make_async_copy(gsrc.at[idxs.at[0]], rows.at[b],
                              gsem.at[b]).wait()

    def start_s(j, b):
        pltpu.async_copy(rows.at[b], acc.at[idxd.at[j]], ssem.at[b],
                         add=True)

    def wait_s(b):
        pltpu.make_async_copy(rows.at[b], acc.at[idxd.at[0]],
                              ssem.at[b]).wait()

    for b in range(_PREF):
        start_g(b, b)

    def body(j0, carry):
        for b in range(_NBUF):
            j = j0 * _NBUF + b
            wait_g(b)
            start_s(j, b)
            jn = j + _PREF
            bn = (b + _PREF) % _NBUF
            if b < _PREF:
                @pl.when(j0 > 0)
                def _():
                    wait_s(bn)
                start_g(jn, bn)
            else:
                @pl.when(j0 + 1 < k_groups)
                def _():
                    wait_s(bn)
                    start_g(jn, bn)
        return carry

    lax.fori_loop(0, k_groups, body, 0)
    for b in range(_NBUF):
        wait_s(b)
    plsc.subcore_barrier()
    pltpu.sync_copy(acc.at[pl.ds(s * rpt, rpt)],
                    out_hbm.at[c, pl.ds(s * rpt, rpt)])


def _dense0_body(degp_ref, x_ref, w0_ref, dinv_ref, g_ref):
    n = x_ref.shape[0]
    deg = degp_ref[0, 0:n, 0:1] + degp_ref[1, 0:n, 0:1] + 1.0
    dinv = lax.rsqrt(jnp.maximum(deg, 1.0))
    dinv_b = jnp.broadcast_to(dinv, (n, _HID))
    dinv_ref[...] = dinv_b
    g_ref[0:n, :] = jnp.dot(x_ref[...], w0_ref[...],
                            preferred_element_type=jnp.float32) * dinv_b


def _mid_body(p_ref, gprev_ref, dinv_ref, w_ref, b_ref, g_ref):
    n = dinv_ref.shape[0]
    dinv = dinv_ref[...]
    agg = p_ref[0, 0:n, :] + p_ref[1, 0:n, :] + gprev_ref[0:n, :]
    h = jnp.maximum(dinv * agg + b_ref[...], 0.0)
    g_ref[0:n, :] = jnp.dot(h, w_ref[...],
                            preferred_element_type=jnp.float32) * dinv


def _final_body(p_ref, gprev_ref, dinv_ref, b_ref, batch_ref, linw_ref,
                linb_ref, out_ref):
    n = dinv_ref.shape[0]
    ng = out_ref.shape[0]
    agg = p_ref[0, 0:n, :] + p_ref[1, 0:n, :] + gprev_ref[0:n, :]
    h = jnp.maximum(dinv_ref[...] * agg + b_ref[...], 0.0)
    gid = lax.broadcasted_iota(jnp.int32, (ng, n), 0)
    onehot = (batch_ref[...] == gid).astype(jnp.float32)
    summed = jnp.dot(onehot, h, preferred_element_type=jnp.float32)
    counts = jnp.sum(onehot, axis=1, keepdims=True)
    pooled = summed / jnp.maximum(counts, 1.0)
    out_ref[...] = (jnp.dot(pooled, linw_ref[...],
                            preferred_element_type=jnp.float32)
                    + linb_ref[...])


def kernel(x, edge_index, batch, W0, b0, W1, b1, W2, b2, W3, b3,
           lin_W, lin_b):
    n = x.shape[0]
    e = edge_index.shape[1]
    ng = 64
    n_pad = ((n + 1 + 127) // 128) * 128
    k_chunks = -(-e // (_NW * _CHUNK))
    k_groups = -(-k_chunks // _NBUF)
    k_chunks = k_groups * _NBUF
    e_pad = k_chunks * _NW * _CHUNK

    src = edge_index[0].astype(jnp.int32)
    dst = edge_index[1].astype(jnp.int32)
    if e_pad > e:
        src = jnp.concatenate([src, jnp.zeros((e_pad - e,), jnp.int32)])
        dst = jnp.concatenate([dst, jnp.full((e_pad - e,), n, jnp.int32)])
    srcw = src.reshape(_NW, k_chunks, _CHUNK)
    dstw = dst.reshape(_NW, k_chunks, _CHUNK)
    zeros_h = jnp.zeros((n_pad, _HID), jnp.float32)
    zeros_d = jnp.zeros((n_pad, _DEGW), jnp.float32)
    ones_d = jnp.ones((_CHUNK, _DEGW), jnp.float32)
    batch2 = batch.astype(jnp.int32).reshape(1, n)

    mesh = plsc.VectorSubcoreMesh(core_axis_name="c", subcore_axis_name="s")
    sc_params = pltpu.CompilerParams(use_tc_tiling_on_sc=False)
    deg_call = pl.kernel(
        functools.partial(_deg_body, k_chunks, n_pad),
        out_type=jax.ShapeDtypeStruct((_NC, n_pad, _DEGW), jnp.float32),
        mesh=mesh,
        scratch_types=[
            pltpu.VMEM((k_chunks, _CHUNK), jnp.int32),
            pltpu.VMEM((_CHUNK, _DEGW), jnp.float32),
            pltpu.VMEM_SHARED((n_pad, _DEGW), jnp.float32),
        ],
        compiler_params=sc_params,
    )
    edge_call = pl.kernel(
        functools.partial(_edge_body, k_groups, n_pad),
        out_type=jax.ShapeDtypeStruct((_NC, n_pad, _HID), jnp.float32),
        mesh=mesh,
        scratch_types=[
            pltpu.VMEM((k_chunks, _CHUNK), jnp.int32),
            pltpu.VMEM((k_chunks, _CHUNK), jnp.int32),
            pltpu.VMEM((_NBUF, _CHUNK, _HID), jnp.float32),
            pltpu.VMEM_SHARED((n_pad, _HID), jnp.float32),
            pltpu.VMEM_SHARED((n_pad, _HID), jnp.float32),
            pltpu.SemaphoreType.DMA((_NBUF,)),
            pltpu.SemaphoreType.DMA((_NBUF,)),
        ],
        compiler_params=sc_params,
    )

    degp = deg_call(dstw, ones_d, zeros_d)

    dinv, g = pl.pallas_call(
        _dense0_body,
        out_shape=(jax.ShapeDtypeStruct((n, _HID), jnp.float32),
                   jax.ShapeDtypeStruct((n_pad, _HID), jnp.float32)),
    )(degp, x, W0)

    for (b_prev, w_next) in ((b0, W1), (b1, W2), (b2, W3)):
        p = edge_call(g, srcw, dstw, zeros_h)
        g = pl.pallas_call(
            _mid_body,
            out_shape=jax.ShapeDtypeStruct((n_pad, _HID), jnp.float32),
        )(p, g, dinv, w_next, b_prev.reshape(1, _HID))

    p = edge_call(g, srcw, dstw, zeros_h)
    out = pl.pallas_call(
        _final_body,
        out_shape=jax.ShapeDtypeStruct((ng, 1), jnp.float32),
    )(p, g, dinv, b3.reshape(1, _HID), batch2, lin_W, lin_b.reshape(1, 1))
    return jnp.squeeze(out, -1)


# speedup vs baseline: 56.7900x; 1.3361x over previous
import functools

import jax
import jax.numpy as jnp
from jax import lax
from jax.experimental import pallas as pl
from jax.experimental.pallas import tpu as pltpu
from jax.experimental.pallas import tpu_sc as plsc

_NC = 2
_NS = 16
_NW = _NC * _NS
_CHUNK = 128
_HID = 32
_PK = 4
_NBUF = 8
_PREF = 4


def _deg_body(k_chunks, n_pad, dstw, ones_hbm, zeros_hbm, out_hbm,
              idxd, ones_v, acc):
    c = lax.axis_index("c")
    s = lax.axis_index("s")
    w = c * _NS + s
    rpt = n_pad // _NS
    pltpu.sync_copy(zeros_hbm.at[pl.ds(s * rpt, rpt)],
                    acc.at[pl.ds(s * rpt, rpt)])
    pltpu.sync_copy(dstw.at[w], idxd)
    pltpu.sync_copy(ones_hbm, ones_v)
    plsc.subcore_barrier()

    def body(j, carry):
        pltpu.sync_copy(ones_v, acc.at[idxd.at[j]], add=True)
        return carry

    lax.fori_loop(0, k_chunks, body, 0)
    plsc.subcore_barrier()
    pltpu.sync_copy(acc.at[pl.ds(s * rpt, rpt)],
                    out_hbm.at[c, pl.ds(s * rpt, rpt)])


def _edge_body(k_groups, n_pad, g_hbm, srcw, dstw, zeros_hbm, out_hbm,
               idxs, idxd, rows, gsrc, acc, gsem, ssem):
    c = lax.axis_index("c")
    s = lax.axis_index("s")
    w = c * _NS + s
    rpt = n_pad // _NS
    pltpu.sync_copy(zeros_hbm.at[pl.ds(s * rpt, rpt)],
                    acc.at[pl.ds(s * rpt, rpt)])
    pltpu.sync_copy(g_hbm.at[pl.ds(s * rpt, rpt)],
                    gsrc.at[pl.ds(s * rpt, rpt)])
    pltpu.sync_copy(srcw.at[w], idxs)
    pltpu.sync_copy(dstw.at[w], idxd)
    plsc.subcore_barrier()

    def start_g(j, b):
        pltpu.async_copy(gsrc.at[idxs.at[j]], rows.at[b], gsem.at[b])

    def wait_g(b):
        pltpu.make_async_copy(gsrc.at[idxs.at[0]], rows.at[b],
                              gsem.at[b]).wait()

    def start_s(j, b):
        pltpu.async_copy(rows.at[b], acc.at[idxd.at[j]], ssem.at[b],
                         add=True)

    def wait_s(b):
        pltpu.make_async_copy(rows.at[b], acc.at[idxd.at[0]],
                              ssem.at[b]).wait()

    for b in range(_PREF):
        start_g(b, b)

    def body(j0, carry):
        for b in range(_NBUF):
            j = j0 * _NBUF + b
            wait_g(b)
            start_s(j, b)
            jn = j + _PREF
            bn = (b + _PREF) % _NBUF
            if b < _PREF:
                @pl.when(j0 > 0)
                def _():
                    wait_s(bn)
                start_g(jn, bn)
            else:
                @pl.when(j0 + 1 < k_groups)
                def _():
                    wait_s(bn)
                    start_g(jn, bn)
        return carry

    lax.fori_loop(0, k_groups, body, 0)
    for b in range(_NBUF):
        wait_s(b)
    plsc.subcore_barrier()
    pltpu.sync_copy(acc.at[pl.ds(s * rpt, rpt)],
                    out_hbm.at[c, pl.ds(s * rpt, rpt)])


def _mm0_body(x_ref, w0_ref, mm_ref):
    mm_ref[...] = jnp.dot(x_ref[...], w0_ref[...],
                          preferred_element_type=jnp.float32)


def _dense0_body(degp_ref, mm4_ref, dinv_ref, g_ref):
    deg = degp_ref[0] + degp_ref[1] + 1.0
    dinv = lax.rsqrt(jnp.maximum(deg, 1.0))
    dinv_ref[...] = dinv
    g_ref[...] = mm4_ref[...] * dinv


def _mid_body(p_ref, gprev_ref, dinv_ref, w4_ref, b4_ref, g_ref):
    dinv = dinv_ref[...]
    agg = p_ref[0] + p_ref[1] + gprev_ref[...]
    h = jnp.maximum(dinv * agg + b4_ref[...], 0.0)
    g_ref[...] = jnp.dot(h, w4_ref[...],
                         preferred_element_type=jnp.float32) * dinv


def _final_body(p_ref, gprev_ref, dinv_ref, b4_ref, batch4_ref, linw_ref,
                linb_ref, out_ref):
    n4 = gprev_ref.shape[0]
    ng = out_ref.shape[0]
    agg = p_ref[0] + p_ref[1] + gprev_ref[...]
    h = jnp.maximum(dinv_ref[...] * agg + b4_ref[...], 0.0)
    gid = lax.broadcasted_iota(jnp.int32, (ng, n4), 0)
    summed = jnp.zeros((ng, _HID), jnp.float32)
    counts = jnp.zeros((ng, 1), jnp.float32)
    for k in range(_PK):
        oh = (batch4_ref[k:k + 1, :] == gid).astype(jnp.float32)
        summed = summed + jnp.dot(oh, h[:, k * _HID:(k + 1) * _HID],
                                  preferred_element_type=jnp.float32)
        counts = counts + jnp.sum(oh, axis=1, keepdims=True)
    pooled = summed / jnp.maximum(counts, 1.0)
    out_ref[...] = (jnp.dot(pooled, linw_ref[...],
                            preferred_element_type=jnp.float32)
                    + linb_ref[...])


def kernel(x, edge_index, batch, W0, b0, W1, b1, W2, b2, W3, b3,
           lin_W, lin_b):
    n = x.shape[0]
    e = edge_index.shape[1]
    ng = 64
    n_pad = ((n + 1 + 511) // 512) * 512
    n4 = n_pad // _PK
    k_chunks = -(-e // (_NW * _CHUNK))
    k_groups = -(-k_chunks // _NBUF)
    k_chunks = k_groups * _NBUF
    e_pad = k_chunks * _NW * _CHUNK

    src = edge_index[0].astype(jnp.int32)
    dst = edge_index[1].astype(jnp.int32)
    if e_pad > e:
        src = jnp.concatenate([src, jnp.zeros((e_pad - e,), jnp.int32)])
        dst = jnp.concatenate([dst, jnp.full((e_pad - e,), n, jnp.int32)])
    srcw = src.reshape(_NW, k_chunks, _CHUNK)
    dstw = dst.reshape(_NW, k_chunks, _CHUNK)
    zeros_h = jnp.zeros((n_pad, _HID), jnp.float32)
    ones_h = jnp.ones((_CHUNK, _HID), jnp.float32)
    batch4 = jnp.pad(batch.astype(jnp.int32).reshape(n // _PK, _PK).T,
                     ((0, 0), (0, n4 - n // _PK)), constant_values=ng)
    def blockdiag(w):
        z = jnp.zeros((_HID, _HID), jnp.float32)
        rows = []
        for i in range(_PK):
            rows.append(jnp.concatenate(
                [w if i == k else z for k in range(_PK)], axis=1))
        return jnp.concatenate(rows, axis=0)

    mesh = plsc.VectorSubcoreMesh(core_axis_name="c", subcore_axis_name="s")
    sc_params = pltpu.CompilerParams(use_tc_tiling_on_sc=False)
    deg_call = pl.kernel(
        functools.partial(_deg_body, k_chunks, n_pad),
        out_type=jax.ShapeDtypeStruct((_NC, n_pad, _HID), jnp.float32),
        mesh=mesh,
        scratch_types=[
            pltpu.VMEM((k_chunks, _CHUNK), jnp.int32),
            pltpu.VMEM((_CHUNK, _HID), jnp.float32),
            pltpu.VMEM_SHARED((n_pad, _HID), jnp.float32),
        ],
        compiler_params=sc_params,
    )
    edge_call = pl.kernel(
        functools.partial(_edge_body, k_groups, n_pad),
        out_type=jax.ShapeDtypeStruct((_NC, n_pad, _HID), jnp.float32),
        mesh=mesh,
        scratch_types=[
            pltpu.VMEM((k_chunks, _CHUNK), jnp.int32),
            pltpu.VMEM((k_chunks, _CHUNK), jnp.int32),
            pltpu.VMEM((_NBUF, _CHUNK, _HID), jnp.float32),
            pltpu.VMEM_SHARED((n_pad, _HID), jnp.float32),
            pltpu.VMEM_SHARED((n_pad, _HID), jnp.float32),
            pltpu.SemaphoreType.DMA((_NBUF,)),
            pltpu.SemaphoreType.DMA((_NBUF,)),
        ],
        compiler_params=sc_params,
    )

    degp = deg_call(dstw, ones_h, zeros_h)
    degp4 = degp.reshape(_NC, n4, _PK * _HID)

    mm = pl.pallas_call(
        _mm0_body,
        out_shape=jax.ShapeDtypeStruct((n, _HID), jnp.float32),
    )(x, W0)
    mm4 = jnp.pad(mm.reshape(n // _PK, _PK * _HID),
                  ((0, n4 - n // _PK), (0, 0)))

    dinv4, g4 = pl.pallas_call(
        _dense0_body,
        out_shape=(jax.ShapeDtypeStruct((n4, _PK * _HID), jnp.float32),
                   jax.ShapeDtypeStruct((n4, _PK * _HID), jnp.float32)),
    )(degp4, mm4)

    for (b_prev, w_next) in ((b0, W1), (b1, W2), (b2, W3)):
        p = edge_call(g4.reshape(n_pad, _HID), srcw, dstw, zeros_h)
        g4 = pl.pallas_call(
            _mid_body,
            out_shape=jax.ShapeDtypeStruct((n4, _PK * _HID), jnp.float32),
        )(p.reshape(_NC, n4, _PK * _HID), g4, dinv4, blockdiag(w_next),
          jnp.tile(b_prev, _PK).reshape(1, _PK * _HID))

    p = edge_call(g4.reshape(n_pad, _HID), srcw, dstw, zeros_h)
    out = pl.pallas_call(
        _final_body,
        out_shape=jax.ShapeDtypeStruct((ng, 1), jnp.float32),
    )(p.reshape(_NC, n4, _PK * _HID), g4, dinv4,
      jnp.tile(b3, _PK).reshape(1, _PK * _HID), batch4, lin_W,
      lin_b.reshape(1, 1))
    return jnp.squeeze(out, -1)
